# baseline (speedup 1.0000x reference)
import jax
import jax.numpy as jnp
import numpy as np
from jax import lax
from jax.experimental import pallas as pl
from jax.experimental.pallas import tpu as pltpu
from jax.experimental.pallas import tpu_sc as plsc

_N = 36
_PERM = tuple(int(v) for v in np.random.RandomState(0).permutation(_N))

_B = 4096
_D = 128
_NC = 2
_NS = 16

_N_TC = 20
_N_SC = _N - _N_TC
_WIN = 256

_SC_J0 = _N_TC
_SC_HALF0 = _N_SC // 2
_SC_HALF1 = _N_SC - _SC_HALF0


def _sc_run(x_hbm, out_hbm, bufs, sems, b0, j0, nslab):
    def start_in(j, b):
        pltpu.async_copy(
            x_hbm.at[_PERM[j0 + j], pl.ds(b0, _WIN), :], bufs[b], sems[b]
        )

    def wait_in(j, b):
        pltpu.make_async_copy(
            x_hbm.at[_PERM[j0 + j], pl.ds(b0, _WIN), :], bufs[b], sems[b]
        ).wait()

    start_in(0, 0)
    if nslab > 1:
        start_in(1, 1)
    for j in range(nslab):
        b = j % 2
        wait_in(j, b)
        pltpu.sync_copy(
            bufs[b], out_hbm.at[j0 + j - _SC_J0, pl.ds(b0, _WIN), :]
        )
        if j + 2 < nslab:
            start_in(j + 2, b)


def _sc_body(x_hbm, out_hbm, buf0, buf1, sem0, sem1):
    c = lax.axis_index("c")
    s = lax.axis_index("s")
    b0 = s * _WIN
    bufs = (buf0, buf1)
    sems = (sem0, sem1)

    @pl.when(c == 0)
    def _():
        _sc_run(x_hbm, out_hbm, bufs, sems, b0, _SC_J0, _SC_HALF0)

    @pl.when(c == 1)
    def _():
        _sc_run(x_hbm, out_hbm, bufs, sems, b0, _SC_J0 + _SC_HALF0, _SC_HALF1)


_TC_BBLK = 4096


def _tc_body(perm_ref, x_ref, o_ref):
    o_ref[...] = x_ref[...]


@jax.jit
def kernel(x):
    xt = jnp.transpose(x, (1, 0, 2))

    perm_tc = jnp.asarray(np.asarray(_PERM[:_N_TC], dtype=np.int32))
    tc_out = pl.pallas_call(
        _tc_body,
        out_shape=jax.ShapeDtypeStruct((_N_TC, _B, _D), x.dtype),
        grid_spec=pltpu.PrefetchScalarGridSpec(
            num_scalar_prefetch=1,
            grid=(_N_TC, _B // _TC_BBLK),
            in_specs=[
                pl.BlockSpec(
                    (1, _TC_BBLK, _D), lambda j, b, perm: (perm[j], b, 0)
                )
            ],
            out_specs=pl.BlockSpec(
                (1, _TC_BBLK, _D), lambda j, b, perm: (j, b, 0)
            ),
        ),
    )(perm_tc, xt)

    if _N_SC > 0:
        mesh = plsc.VectorSubcoreMesh(core_axis_name="c", subcore_axis_name="s")
        sc_out = pl.kernel(
            _sc_body,
            out_type=jax.ShapeDtypeStruct((_N_SC, _B, _D), x.dtype),
            mesh=mesh,
            scratch_types=[
                pltpu.VMEM((_WIN, _D), jnp.float32),
                pltpu.VMEM((_WIN, _D), jnp.float32),
                pltpu.SemaphoreType.DMA,
                pltpu.SemaphoreType.DMA,
            ],
        )(xt)
        out_t = jnp.concatenate([tc_out, sc_out], axis=0)
    else:
        out_t = tc_out
    return jnp.transpose(out_t, (1, 0, 2))


# speedup vs baseline: 1.5843x; 1.5843x over previous
import jax
import jax.numpy as jnp
import numpy as np
from jax import lax
from jax.experimental import pallas as pl
from jax.experimental.pallas import tpu as pltpu
from jax.experimental.pallas import tpu_sc as plsc

_N = 36
_PERM = tuple(int(v) for v in np.random.RandomState(0).permutation(_N))

_B = 4096
_D = 128
_NC = 2
_NS = 16
_WIN = 256
_HALF = _N // 2
_NBUF = 3


def _run(x_hbm, out_hbm, bufs, semr, semw, b0, j0):
    def start_in(j, b):
        pltpu.async_copy(
            x_hbm.at[_PERM[j0 + j], pl.ds(b0, _WIN), :], bufs[b], semr[b]
        )

    def wait_in(j, b):
        pltpu.make_async_copy(
            x_hbm.at[_PERM[j0 + j], pl.ds(b0, _WIN), :], bufs[b], semr[b]
        ).wait()

    def start_out(j, b):
        pltpu.async_copy(
            bufs[b], out_hbm.at[j0 + j, pl.ds(b0, _WIN), :], semw[b]
        )

    def wait_out(j, b):
        pltpu.make_async_copy(
            bufs[b], out_hbm.at[j0 + j, pl.ds(b0, _WIN), :], semw[b]
        ).wait()

    for b in range(_NBUF):
        start_in(b, b)

    for t in range(_HALF):
        b = t % _NBUF
        wait_in(t, b)
        start_out(t, b)
        r = t + 1
        if _NBUF <= r < _HALF:
            rb = r % _NBUF
            wait_out(r - _NBUF, rb)
            start_in(r, rb)

    for t in range(_HALF - _NBUF, _HALF):
        wait_out(t, t % _NBUF)


def _body(x_hbm, out_hbm, buf0, buf1, buf2, semr0, semr1, semr2,
          semw0, semw1, semw2):
    c = lax.axis_index("c")
    s = lax.axis_index("s")
    b0 = s * _WIN
    bufs = (buf0, buf1, buf2)
    semr = (semr0, semr1, semr2)
    semw = (semw0, semw1, semw2)

    @pl.when(c == 0)
    def _():
        _run(x_hbm, out_hbm, bufs, semr, semw, b0, 0)

    @pl.when(c == 1)
    def _():
        _run(x_hbm, out_hbm, bufs, semr, semw, b0, _HALF)


@jax.jit
def kernel(x):
    xt = jnp.transpose(x, (1, 0, 2))
    mesh = plsc.VectorSubcoreMesh(core_axis_name="c", subcore_axis_name="s")
    out_t = pl.kernel(
        _body,
        out_type=jax.ShapeDtypeStruct((_N, _B, _D), x.dtype),
        mesh=mesh,
        scratch_types=[
            pltpu.VMEM((_WIN, _D), jnp.float32),
            pltpu.VMEM((_WIN, _D), jnp.float32),
            pltpu.VMEM((_WIN, _D), jnp.float32),
            pltpu.SemaphoreType.DMA,
            pltpu.SemaphoreType.DMA,
            pltpu.SemaphoreType.DMA,
            pltpu.SemaphoreType.DMA,
            pltpu.SemaphoreType.DMA,
            pltpu.SemaphoreType.DMA,
        ],
    )(xt)
    return jnp.transpose(out_t, (1, 0, 2))
